# baseline (speedup 1.0000x reference)
import functools

import jax
import jax.numpy as jnp
from jax import lax
from jax.experimental import pallas as pl
from jax.experimental.pallas import tpu as pltpu
from jax.experimental.pallas import tpu_sc as plsc

EMBED = 64
SEQ = 200
BATCH = 1024
CHUNK = 100
NW = 32
CHUNKS = (BATCH * SEQ) // (NW * CHUNK)
LANES = 16


def _emb_body(ids_hbm, table_hbm, pos_hbm, out_hbm, idx_v, pos_v, buf_v, sem):
    c = lax.axis_index("c")
    s = lax.axis_index("s")
    wid = s * 2 + c

    pltpu.sync_copy(ids_hbm.at[wid], idx_v)
    pltpu.sync_copy(pos_hbm.at[pl.ds(0, SEQ)], pos_v)

    def body(g, carry):
        pltpu.async_copy(table_hbm.at[idx_v.at[g]], buf_v, sem).wait()
        half = lax.rem(g, 2) * CHUNK

        def add_row(j, c2):
            for k in range(EMBED // LANES):
                sl = pl.ds(k * LANES, LANES)
                buf_v[j, sl] = buf_v[j, sl] + pos_v[half + j, sl]
            return c2

        lax.fori_loop(0, CHUNK, add_row, 0)
        pltpu.sync_copy(buf_v, out_hbm.at[wid, g])
        return carry

    lax.fori_loop(0, CHUNKS, body, 0)


@jax.jit
def _emb(ids, table, pos):
    mesh = plsc.VectorSubcoreMesh(core_axis_name="c", subcore_axis_name="s")
    f = functools.partial(
        pl.kernel,
        mesh=mesh,
        out_type=jax.ShapeDtypeStruct((NW, CHUNKS, CHUNK, EMBED), jnp.float32),
        scratch_types=[
            pltpu.VMEM((CHUNKS, CHUNK), jnp.int32),
            pltpu.VMEM((SEQ, EMBED), jnp.float32),
            pltpu.VMEM((CHUNK, EMBED), jnp.float32),
            pltpu.SemaphoreType.DMA,
        ],
        compiler_params=pltpu.CompilerParams(use_tc_tiling_on_sc=False),
    )(_emb_body)
    return f(ids, table, pos)


def kernel(token_ids, token_table, pos_table):
    ids = token_ids.reshape(NW, CHUNKS, CHUNK)
    out = _emb(ids, token_table, pos_table)
    return out.reshape(BATCH, SEQ, EMBED)


# speedup vs baseline: 1.2258x; 1.2258x over previous
import functools

import jax
import jax.numpy as jnp
from jax import lax
from jax.experimental import pallas as pl
from jax.experimental.pallas import tpu as pltpu
from jax.experimental.pallas import tpu_sc as plsc

EMBED = 64
SEQ = 200
BATCH = 1024
CHUNK = 100
NW = 32
CHUNKS = (BATCH * SEQ) // (NW * CHUNK)
LANES = 16
NBUF = 4
QUADS = CHUNKS // NBUF


def _emb_body(ids_hbm, table_hbm, pos_hbm, out_hbm, idx_v, pos_v, bufs, gsems, ssems):
    c = lax.axis_index("c")
    s = lax.axis_index("s")
    wid = s * 2 + c

    pltpu.sync_copy(ids_hbm.at[wid], idx_v)
    pltpu.sync_copy(pos_hbm.at[pl.ds(0, SEQ)], pos_v)

    def start_gather(g, b):
        pltpu.async_copy(table_hbm.at[idx_v.at[g]], bufs[b], gsems[b])

    def wait_gather(b):
        pltpu.make_async_copy(table_hbm.at[idx_v.at[0]], bufs[b], gsems[b]).wait()

    def start_store(g, b):
        pltpu.async_copy(bufs[b], out_hbm.at[wid, g], ssems[b])

    def wait_store(b):
        pltpu.make_async_copy(bufs[b], out_hbm.at[wid, 0], ssems[b]).wait()

    def add_pos(b, half):
        buf = bufs[b]

        def add_row(j, c2):
            for jj in range(2):
                for k in range(EMBED // LANES):
                    sl = pl.ds(k * LANES, LANES)
                    plsc.addupdate(buf.at[2 * j + jj, sl], pos_v[half + 2 * j + jj, sl])
            return c2

        lax.fori_loop(0, CHUNK // 2, add_row, 0)

    start_gather(0, 0)
    start_gather(1, 1)

    def quad(q, carry):
        for i in range(NBUF):
            g = NBUF * q + i
            b2 = (i + 2) % NBUF
            if i < 2:
                @pl.when(q >= 1)
                def _():
                    wait_store(b2)
                    start_gather(g + 2, b2)

                @pl.when(q < 1)
                def _():
                    start_gather(g + 2, b2)
            else:
                wait_store(b2)

                @pl.when(q < QUADS - 1)
                def _():
                    start_gather(g + 2, b2)

            wait_gather(i)
            add_pos(i, (i % 2) * CHUNK)
            start_store(g, i)
        return carry

    lax.fori_loop(0, QUADS, quad, 0)
    wait_store(2)
    wait_store(3)


@jax.jit
def _emb(ids, table, pos):
    mesh = plsc.VectorSubcoreMesh(core_axis_name="c", subcore_axis_name="s")
    f = functools.partial(
        pl.kernel,
        mesh=mesh,
        out_type=jax.ShapeDtypeStruct((NW, CHUNKS, CHUNK, EMBED), jnp.float32),
        scratch_types=[
            pltpu.VMEM((CHUNKS, CHUNK), jnp.int32),
            pltpu.VMEM((SEQ, EMBED), jnp.float32),
            [pltpu.VMEM((CHUNK, EMBED), jnp.float32) for _ in range(NBUF)],
            [pltpu.SemaphoreType.DMA for _ in range(NBUF)],
            [pltpu.SemaphoreType.DMA for _ in range(NBUF)],
        ],
        compiler_params=pltpu.CompilerParams(use_tc_tiling_on_sc=False),
    )(_emb_body)
    return f(ids, table, pos)


def kernel(token_ids, token_table, pos_table):
    ids = token_ids.reshape(NW, CHUNKS, CHUNK)
    out = _emb(ids, token_table, pos_table)
    return out.reshape(BATCH, SEQ, EMBED)


# speedup vs baseline: 1.2273x; 1.0012x over previous
import functools

import jax
import jax.numpy as jnp
from jax import lax
from jax.experimental import pallas as pl
from jax.experimental.pallas import tpu as pltpu
from jax.experimental.pallas import tpu_sc as plsc

EMBED = 64
SEQ = 200
BATCH = 1024
CHUNK = 100
NW = 32
BPW = BATCH // NW
LANES = 16
NBUF = 4
QUADS = BPW // NBUF


def _emb_body(ids_hbm, table_hbm, pos_hbm, out_hbm, idx_v, pos_v, bufs, gsems, ssems):
    c = lax.axis_index("c")
    s = lax.axis_index("s")
    wid = s * 2 + c
    b0 = wid * BPW

    pltpu.sync_copy(ids_hbm.at[pl.ds(b0, BPW)], idx_v)
    pltpu.sync_copy(pos_hbm.at[pl.ds(0, SEQ)], pos_v)

    def start_gather(bl, b):
        for off, n in ((0, 104), (104, 96)):
            pltpu.async_copy(
                table_hbm.at[idx_v.at[bl, pl.ds(off, n)]],
                bufs[b].at[pl.ds(off, n)],
                gsems[b],
            )

    def wait_gather(b):
        pltpu.make_async_copy(
            table_hbm.at[idx_v.at[0, pl.ds(0, CHUNK)]], bufs[b], gsems[b]
        ).wait()

    def start_store(bl, b):
        pltpu.async_copy(bufs[b], out_hbm.at[b0 + bl], ssems[b])

    def wait_store(b):
        pltpu.make_async_copy(bufs[b], out_hbm.at[0], ssems[b]).wait()

    def add_pos(b):
        buf = bufs[b]

        def add_row(j, c2):
            for jj in range(2):
                for k in range(EMBED // LANES):
                    sl = pl.ds(k * LANES, LANES)
                    plsc.addupdate(buf.at[2 * j + jj, sl], pos_v[2 * j + jj, sl])
            return c2

        lax.fori_loop(0, SEQ // 2, add_row, 0)

    start_gather(0, 0)
    start_gather(1, 1)

    def quad(q, carry):
        for i in range(NBUF):
            bl = NBUF * q + i
            b2 = (i + 2) % NBUF
            if i < 2:
                @pl.when(q >= 1)
                def _():
                    wait_store(b2)
                    start_gather(bl + 2, b2)

                @pl.when(q < 1)
                def _():
                    start_gather(bl + 2, b2)
            else:
                wait_store(b2)

                @pl.when(q < QUADS - 1)
                def _():
                    start_gather(bl + 2, b2)

            wait_gather(i)
            add_pos(i)
            start_store(bl, i)
        return carry

    lax.fori_loop(0, QUADS, quad, 0)
    wait_store(2)
    wait_store(3)


@jax.jit
def _emb(ids, table, pos):
    mesh = plsc.VectorSubcoreMesh(core_axis_name="c", subcore_axis_name="s")
    f = functools.partial(
        pl.kernel,
        mesh=mesh,
        out_type=jax.ShapeDtypeStruct((BATCH, SEQ, EMBED), jnp.float32),
        scratch_types=[
            pltpu.VMEM((BPW, SEQ), jnp.int32),
            pltpu.VMEM((SEQ, EMBED), jnp.float32),
            [pltpu.VMEM((SEQ, EMBED), jnp.float32) for _ in range(NBUF)],
            [pltpu.SemaphoreType.DMA for _ in range(NBUF)],
            [pltpu.SemaphoreType.DMA for _ in range(NBUF)],
        ],
        compiler_params=pltpu.CompilerParams(use_tc_tiling_on_sc=False),
    )(_emb_body)
    return f(ids, table, pos)


def kernel(token_ids, token_table, pos_table):
    return _emb(token_ids, token_table, pos_table)
